# Initial kernel scaffold; baseline (speedup 1.0000x reference)
import functools

import jax
import jax.numpy as jnp
from jax import lax
from jax.experimental import pallas as pl
from jax.experimental.pallas import tpu as pltpu
from jax.experimental.pallas import tpu_sc as plsc

N_NODES = 10000
N_EDGES = 320000
D = 128

NC = 2
NS = 16
NW = NC * NS
LANES = 16

EDGE_BATCH = 128
NUM_BATCHES = N_EDGES // EDGE_BATCH
ROWS_PER_SUBCORE = N_NODES // NS
ZERO_ROWS = 125


def _matmul_body(x_ref, w_ref, o_ref):
    o_ref[...] = jnp.dot(x_ref[...], w_ref[...],
                         preferred_element_type=jnp.float32)


def _tc_matmul(x, w):
    return pl.pallas_call(
        _matmul_body,
        out_shape=jax.ShapeDtypeStruct((N_NODES, D), jnp.float32),
    )(x, w)


def _combine_body(p_ref, b_ref, o_ref):
    o_ref[...] = p_ref[0] + p_ref[1] + b_ref[...]


def _tc_combine(partials, bias2d):
    return pl.pallas_call(
        _combine_body,
        out_shape=jax.ShapeDtypeStruct((N_NODES, D), jnp.float32),
    )(partials, bias2d)


def _spmm_sc(support, src, dst, vals):
    mesh = plsc.VectorSubcoreMesh(core_axis_name="c", subcore_axis_name="s")

    @functools.partial(
        pl.kernel,
        out_type=jax.ShapeDtypeStruct((NC, N_NODES, D), jnp.float32),
        mesh=mesh,
        scratch_types=[
            pltpu.VMEM((EDGE_BATCH,), jnp.int32),
            pltpu.VMEM((EDGE_BATCH,), jnp.int32),
            pltpu.VMEM((EDGE_BATCH,), jnp.float32),
            pltpu.VMEM((EDGE_BATCH, D), jnp.float32),
            pltpu.VMEM((ZERO_ROWS, D), jnp.float32),
            pltpu.VMEM_SHARED((N_NODES, D), jnp.float32),
        ],
    )
    def k(support_hbm, src_hbm, dst_hbm, vals_hbm, out_hbm,
          src_v, dst_v, vals_v, rows_v, zero_v, acc_sh):
        cid = lax.axis_index("c")
        sid = lax.axis_index("s")
        wid = cid * NS + sid

        zvec = jnp.zeros((LANES,), jnp.float32)

        @pl.loop(0, ZERO_ROWS)
        def _(i):
            for c in range(D // LANES):
                zero_v[i, pl.ds(c * LANES, LANES)] = zvec

        for kk in range(ROWS_PER_SUBCORE // ZERO_ROWS):
            base = sid * ROWS_PER_SUBCORE + kk * ZERO_ROWS
            pltpu.sync_copy(zero_v, acc_sh.at[pl.ds(base, ZERO_ROWS)])

        plsc.subcore_barrier()

        @pl.loop(wid, NUM_BATCHES, step=NW)
        def _(b):
            off = b * EDGE_BATCH
            pltpu.sync_copy(src_hbm.at[pl.ds(off, EDGE_BATCH)], src_v)
            pltpu.sync_copy(dst_hbm.at[pl.ds(off, EDGE_BATCH)], dst_v)
            pltpu.sync_copy(vals_hbm.at[pl.ds(off, EDGE_BATCH)], vals_v)
            pltpu.sync_copy(support_hbm.at[src_v], rows_v)

            @pl.loop(0, EDGE_BATCH)
            def _(e):
                v = vals_v[e]
                for c in range(D // LANES):
                    sl = pl.ds(c * LANES, LANES)
                    rows_v[e, sl] = rows_v[e, sl] * v

            pltpu.sync_copy(rows_v, acc_sh.at[dst_v], add=True)

        plsc.subcore_barrier()

        dbase = sid * ROWS_PER_SUBCORE
        pltpu.sync_copy(acc_sh.at[pl.ds(dbase, ROWS_PER_SUBCORE)],
                        out_hbm.at[cid, pl.ds(dbase, ROWS_PER_SUBCORE)])

    return k(support, src, dst, vals)


def kernel(sparse_poly_edge_index, sparse_poly_values, input_feature,
           weight, bias):
    dst = sparse_poly_edge_index[0].astype(jnp.int32)
    src = sparse_poly_edge_index[1].astype(jnp.int32)
    support = _tc_matmul(input_feature, weight)
    partials = _spmm_sc(support, src, dst, sparse_poly_values)
    return _tc_combine(partials, bias.reshape(1, D))


# speedup vs baseline: 5.4741x; 5.4741x over previous
import functools

import jax
import jax.numpy as jnp
from jax import lax
from jax.experimental import pallas as pl
from jax.experimental.pallas import tpu as pltpu
from jax.experimental.pallas import tpu_sc as plsc

N_NODES = 10000
N_EDGES = 320000
D = 128

NC = 2
NS = 16
NW = NC * NS
LANES = 16

EDGE_BATCH = 128
NUM_BATCHES = N_EDGES // EDGE_BATCH
ROW_CHUNK = 200
NUM_ROW_CHUNKS = N_NODES // ROW_CHUNK


def _matmul_body(x_ref, w_ref, o_ref):
    o_ref[...] = jnp.dot(x_ref[...], w_ref[...],
                         preferred_element_type=jnp.float32)


def _tc_matmul(x, w):
    return pl.pallas_call(
        _matmul_body,
        out_shape=jax.ShapeDtypeStruct((N_NODES, D), jnp.float32),
    )(x, w)


def _combine_body(p_ref, b_ref, o_ref):
    o_ref[...] = p_ref[0] + p_ref[1] + b_ref[...]


def _tc_combine(partials, bias2d):
    return pl.pallas_call(
        _combine_body,
        out_shape=jax.ShapeDtypeStruct((N_NODES, D), jnp.float32),
    )(partials, bias2d)


def _spmm_sc(support, src, dst, vals):
    mesh = plsc.VectorSubcoreMesh(core_axis_name="c", subcore_axis_name="s")

    @functools.partial(
        pl.kernel,
        out_type=jax.ShapeDtypeStruct((NC, N_NODES, D), jnp.float32),
        mesh=mesh,
        scratch_types=[
            pltpu.VMEM((EDGE_BATCH,), jnp.int32),
            pltpu.VMEM((EDGE_BATCH,), jnp.int32),
            pltpu.VMEM((EDGE_BATCH,), jnp.float32),
            pltpu.VMEM((EDGE_BATCH, D), jnp.float32),
            pltpu.VMEM((ROW_CHUNK, D), jnp.float32),
            pltpu.VMEM_SHARED((N_NODES, D), jnp.float32),
        ],
    )
    def k(support_hbm, src_hbm, dst_hbm, vals_hbm, out_hbm,
          src_v, dst_v, vals_v, rows_v, zero_v, acc_sh):
        cid = lax.axis_index("c")
        sid = lax.axis_index("s")
        wid = cid * NS + sid

        zvec = jnp.zeros((LANES,), jnp.float32)

        @pl.loop(0, ROW_CHUNK)
        def _(i):
            for c in range(D // LANES):
                zero_v[i, pl.ds(c * LANES, LANES)] = zvec

        @pl.loop(sid, NUM_ROW_CHUNKS, step=NS)
        def _(r):
            pltpu.sync_copy(zero_v, acc_sh.at[pl.ds(r * ROW_CHUNK, ROW_CHUNK)])

        plsc.subcore_barrier()

        @pl.loop(wid, NUM_BATCHES, step=NW)
        def _(b):
            off = b * EDGE_BATCH
            pltpu.sync_copy(src_hbm.at[pl.ds(off, EDGE_BATCH)], src_v)
            pltpu.sync_copy(dst_hbm.at[pl.ds(off, EDGE_BATCH)], dst_v)
            pltpu.sync_copy(vals_hbm.at[pl.ds(off, EDGE_BATCH)], vals_v)
            pltpu.sync_copy(support_hbm.at[src_v], rows_v)

            @pl.loop(0, EDGE_BATCH, step=LANES)
            def _(g):
                vvec = vals_v[pl.ds(g, LANES)]
                for i in range(LANES):
                    v = vvec[i]
                    for c in range(D // LANES):
                        sl = pl.ds(c * LANES, LANES)
                        rows_v[g + i, sl] = rows_v[g + i, sl] * v

            pltpu.sync_copy(rows_v, acc_sh.at[dst_v], add=True)

        plsc.subcore_barrier()

        @pl.loop(sid, NUM_ROW_CHUNKS, step=NS)
        def _(r):
            dbase = r * ROW_CHUNK
            pltpu.sync_copy(acc_sh.at[pl.ds(dbase, ROW_CHUNK)],
                            out_hbm.at[cid, pl.ds(dbase, ROW_CHUNK)])

    return k(support, src, dst, vals)


def kernel(sparse_poly_edge_index, sparse_poly_values, input_feature,
           weight, bias):
    dst = sparse_poly_edge_index[0].astype(jnp.int32)
    src = sparse_poly_edge_index[1].astype(jnp.int32)
    support = _tc_matmul(input_feature, weight)
    partials = _spmm_sc(support, src, dst, sparse_poly_values)
    return _tc_combine(partials, bias.reshape(1, D))


# speedup vs baseline: 9.5344x; 1.7417x over previous
import functools

import jax
import jax.numpy as jnp
from jax import lax
from jax.experimental import pallas as pl
from jax.experimental.pallas import tpu as pltpu
from jax.experimental.pallas import tpu_sc as plsc

N_NODES = 10000
N_EDGES = 320000
D = 128

NC = 2
NS = 16
NW = NC * NS
LANES = 16

B = 80
BATCHES_PER_WORKER = N_EDGES // (NW * B)
WINDOW = 25
NUM_WINDOWS = BATCHES_PER_WORKER // WINDOW
ROW_CHUNK = 80
NUM_ROW_CHUNKS = N_NODES // ROW_CHUNK


def _matmul_body(x_ref, w_ref, o_ref):
    o_ref[...] = jnp.dot(x_ref[...], w_ref[...],
                         preferred_element_type=jnp.float32)


def _tc_matmul(x, w):
    return pl.pallas_call(
        _matmul_body,
        out_shape=jax.ShapeDtypeStruct((N_NODES, D), jnp.float32),
    )(x, w)


def _combine_body(p_ref, b_ref, o_ref):
    o_ref[...] = p_ref[0] + p_ref[1] + b_ref[...]


def _tc_combine(partials, bias2d):
    return pl.pallas_call(
        _combine_body,
        out_shape=jax.ShapeDtypeStruct((N_NODES, D), jnp.float32),
    )(partials, bias2d)


def _spmm_sc(support, src3, dst3, vals3):
    mesh = plsc.VectorSubcoreMesh(core_axis_name="c", subcore_axis_name="s")

    @functools.partial(
        pl.kernel,
        out_type=jax.ShapeDtypeStruct((NC, N_NODES, D), jnp.float32),
        mesh=mesh,
        scratch_types=[
            pltpu.VMEM((WINDOW, B), jnp.int32),
            pltpu.VMEM((WINDOW, B), jnp.int32),
            pltpu.VMEM((WINDOW, B), jnp.float32),
            pltpu.VMEM((B, D), jnp.float32),
            pltpu.VMEM((B, D), jnp.float32),
            pltpu.VMEM_SHARED((N_NODES, D), jnp.float32),
            pltpu.SemaphoreType.DMA,
            pltpu.SemaphoreType.DMA,
            pltpu.SemaphoreType.DMA,
        ],
    )
    def k(support_hbm, src_hbm, dst_hbm, vals_hbm, out_hbm,
          src_v, dst_v, vals_v, rows0, rows1, acc_sh, sem_i, sem0, sem1):
        cid = lax.axis_index("c")
        sid = lax.axis_index("s")
        wid = cid * NS + sid

        zvec = jnp.zeros((LANES,), jnp.float32)

        @pl.loop(0, B)
        def _(i):
            for c in range(D // LANES):
                rows0[i, pl.ds(c * LANES, LANES)] = zvec

        @pl.loop(sid, NUM_ROW_CHUNKS, step=NS)
        def _(r):
            pltpu.sync_copy(rows0, acc_sh.at[pl.ds(r * ROW_CHUNK, ROW_CHUNK)])

        plsc.subcore_barrier()

        def scale_and_scatter(buf, j):
            @pl.loop(0, B, step=LANES)
            def _(g):
                vvec = vals_v[j, pl.ds(g, LANES)]
                for i in range(LANES):
                    v = vvec[i]
                    for c in range(D // LANES):
                        sl = pl.ds(c * LANES, LANES)
                        buf[g + i, sl] = buf[g + i, sl] * v

            pltpu.sync_copy(buf, acc_sh.at[dst_v.at[j]], add=True)

        def wait_gather(buf, sem):
            pltpu.make_async_copy(support_hbm.at[pl.ds(0, B)], buf,
                                  sem).wait()

        @pl.loop(0, NUM_WINDOWS)
        def _(w):
            pltpu.sync_copy(src_hbm.at[wid, w], src_v)
            pltpu.sync_copy(dst_hbm.at[wid, w], dst_v)
            pltpu.sync_copy(vals_hbm.at[wid, w], vals_v)

            pltpu.async_copy(support_hbm.at[src_v.at[0]], rows0, sem0)

            @pl.loop(0, (WINDOW - 1) // 2)
            def _(i):
                j0 = 2 * i
                wait_gather(rows0, sem0)
                pltpu.async_copy(support_hbm.at[src_v.at[j0 + 1]], rows1,
                                 sem1)
                scale_and_scatter(rows0, j0)
                wait_gather(rows1, sem1)
                pltpu.async_copy(support_hbm.at[src_v.at[j0 + 2]], rows0,
                                 sem0)
                scale_and_scatter(rows1, j0 + 1)

            wait_gather(rows0, sem0)
            scale_and_scatter(rows0, WINDOW - 1)

        plsc.subcore_barrier()

        @pl.loop(sid, NUM_ROW_CHUNKS, step=NS)
        def _(r):
            dbase = r * ROW_CHUNK
            pltpu.sync_copy(acc_sh.at[pl.ds(dbase, ROW_CHUNK)],
                            out_hbm.at[cid, pl.ds(dbase, ROW_CHUNK)])

    return k(support, src3, dst3, vals3)


def kernel(sparse_poly_edge_index, sparse_poly_values, input_feature,
           weight, bias):
    dst = sparse_poly_edge_index[0].astype(jnp.int32)
    src = sparse_poly_edge_index[1].astype(jnp.int32)
    src3 = src.reshape(NW, NUM_WINDOWS, WINDOW, B)
    dst3 = dst.reshape(NW, NUM_WINDOWS, WINDOW, B)
    vals3 = sparse_poly_values.reshape(NW, NUM_WINDOWS, WINDOW, B)
    support = _tc_matmul(input_feature, weight)
    partials = _spmm_sc(support, src3, dst3, vals3)
    return _tc_combine(partials, bias.reshape(1, D))


# speedup vs baseline: 11.0413x; 1.1581x over previous
import functools

import jax
import jax.numpy as jnp
from jax import lax
from jax.experimental import pallas as pl
from jax.experimental.pallas import tpu as pltpu
from jax.experimental.pallas import tpu_sc as plsc

N_NODES = 10000
N_EDGES = 320000
D = 128

NC = 2
NS = 16
NW = NC * NS
LANES = 16

B = 80
BATCHES_PER_WORKER = N_EDGES // (NW * B)
WINDOW = 25
NUM_WINDOWS = BATCHES_PER_WORKER // WINDOW
ROW_CHUNK = 80
NUM_ROW_CHUNKS = N_NODES // ROW_CHUNK


def _matmul_body(x_ref, w_ref, o_ref):
    o_ref[...] = jnp.dot(x_ref[...], w_ref[...],
                         preferred_element_type=jnp.float32)


def _tc_matmul(x, w):
    return pl.pallas_call(
        _matmul_body,
        out_shape=jax.ShapeDtypeStruct((N_NODES, D), jnp.float32),
    )(x, w)


def _combine_body(p_ref, b_ref, o_ref):
    o_ref[...] = p_ref[0] + p_ref[1] + b_ref[...]


def _tc_combine(partials, bias2d):
    return pl.pallas_call(
        _combine_body,
        out_shape=jax.ShapeDtypeStruct((N_NODES, D), jnp.float32),
    )(partials, bias2d)


def _spmm_sc(support, src3, dst3, vals3):
    mesh = plsc.VectorSubcoreMesh(core_axis_name="c", subcore_axis_name="s")

    @functools.partial(
        pl.kernel,
        out_type=jax.ShapeDtypeStruct((NC, N_NODES, D), jnp.float32),
        mesh=mesh,
        scratch_types=[
            pltpu.VMEM((WINDOW, B), jnp.int32),
            pltpu.VMEM((WINDOW, B), jnp.int32),
            pltpu.VMEM((WINDOW, B), jnp.float32),
            pltpu.VMEM((B, D), jnp.float32),
            pltpu.VMEM((B, D), jnp.float32),
            pltpu.VMEM((B, D), jnp.float32),
            pltpu.VMEM_SHARED((N_NODES, D), jnp.float32),
            pltpu.SemaphoreType.DMA,
            pltpu.SemaphoreType.DMA,
            pltpu.SemaphoreType.DMA,
            pltpu.SemaphoreType.DMA,
            pltpu.SemaphoreType.DMA,
            pltpu.SemaphoreType.DMA,
        ],
    )
    def k(support_hbm, src_hbm, dst_hbm, vals_hbm, out_hbm,
          src_v, dst_v, vals_v, rows0, rows1, rows2, acc_sh,
          sg0, sg1, sg2, ss0, ss1, ss2):
        cid = lax.axis_index("c")
        sid = lax.axis_index("s")
        wid = cid * NS + sid

        zvec = jnp.zeros((LANES,), jnp.float32)

        @pl.loop(0, B)
        def _(i):
            for c in range(D // LANES):
                rows0[i, pl.ds(c * LANES, LANES)] = zvec

        @pl.loop(sid, NUM_ROW_CHUNKS, step=NS)
        def _(r):
            pltpu.sync_copy(rows0, acc_sh.at[pl.ds(r * ROW_CHUNK, ROW_CHUNK)])

        plsc.subcore_barrier()

        rows = (rows0, rows1, rows2)
        sg = (sg0, sg1, sg2)
        ss = (ss0, ss1, ss2)

        def scale(buf, j):
            @pl.loop(0, B, step=LANES)
            def _(g):
                vvec = vals_v[j, pl.ds(g, LANES)]
                for i in range(LANES):
                    v = vvec[i]
                    for c in range(D // LANES):
                        sl = pl.ds(c * LANES, LANES)
                        buf[g + i, sl] = buf[g + i, sl] * v

        def start_gather(j, b):
            pltpu.async_copy(support_hbm.at[src_v.at[j]], rows[b], sg[b])

        def wait_gather(b):
            pltpu.make_async_copy(support_hbm.at[pl.ds(0, B)], rows[b],
                                  sg[b]).wait()

        def start_scatter(j, b):
            pltpu.async_copy(rows[b], acc_sh.at[dst_v.at[j]], ss[b],
                             add=True)

        def wait_scatter(b):
            pltpu.make_async_copy(rows[b], acc_sh.at[pl.ds(0, B)],
                                  ss[b]).wait()

        @pl.loop(0, NUM_WINDOWS)
        def _(w):
            pltpu.sync_copy(src_hbm.at[wid, w], src_v)
            pltpu.sync_copy(dst_hbm.at[wid, w], dst_v)
            pltpu.sync_copy(vals_hbm.at[wid, w], vals_v)

            start_gather(0, 0)
            start_gather(1, 1)
            start_gather(2, 2)

            wait_gather(0)
            scale(rows0, 0)
            start_scatter(0, 0)

            @pl.loop(0, (WINDOW - 1) // 3)
            def _(i):
                jb = 1 + 3 * i
                for t in range(3):
                    b = (1 + t) % 3
                    wait_gather(b)
                    scale(rows[b], jb + t)
                    start_scatter(jb + t, b)
                    nxt = jb + t + 2
                    bn = t

                    @pl.when(nxt < WINDOW)
                    def _():
                        wait_scatter(bn)
                        start_gather(nxt, bn)

            wait_scatter((WINDOW - 3) % 3)
            wait_scatter((WINDOW - 2) % 3)
            wait_scatter((WINDOW - 1) % 3)

        plsc.subcore_barrier()

        @pl.loop(sid, NUM_ROW_CHUNKS, step=NS)
        def _(r):
            dbase = r * ROW_CHUNK
            pltpu.sync_copy(acc_sh.at[pl.ds(dbase, ROW_CHUNK)],
                            out_hbm.at[cid, pl.ds(dbase, ROW_CHUNK)])

    return k(support, src3, dst3, vals3)


def kernel(sparse_poly_edge_index, sparse_poly_values, input_feature,
           weight, bias):
    dst = sparse_poly_edge_index[0].astype(jnp.int32)
    src = sparse_poly_edge_index[1].astype(jnp.int32)
    src3 = src.reshape(NW, NUM_WINDOWS, WINDOW, B)
    dst3 = dst.reshape(NW, NUM_WINDOWS, WINDOW, B)
    vals3 = sparse_poly_values.reshape(NW, NUM_WINDOWS, WINDOW, B)
    support = _tc_matmul(input_feature, weight)
    partials = _spmm_sc(support, src3, dst3, vals3)
    return _tc_combine(partials, bias.reshape(1, D))
